# baseline (speedup 1.0000x reference)
import functools

import jax
import jax.numpy as jnp
from jax import lax
from jax.experimental import pallas as pl
from jax.experimental.pallas import tpu as pltpu
from jax.experimental.pallas import tpu_sc as plsc

DIM = 64
CHUNK = 128


@functools.cache
def _make_gather(num_idx: int, dim: int):
    info = plsc.get_sparse_core_info()
    nw = info.num_cores * info.num_subcores
    assert num_idx % (nw * CHUNK) == 0
    chunks_per_w = num_idx // (nw * CHUNK)
    mesh = plsc.VectorSubcoreMesh(core_axis_name="c", subcore_axis_name="s")

    @functools.partial(
        pl.kernel,
        mesh=mesh,
        out_type=jax.ShapeDtypeStruct((num_idx, dim), jnp.float32),
        scratch_types=[
            pltpu.VMEM((chunks_per_w, CHUNK), jnp.int32),
            pltpu.VMEM((2, CHUNK, dim), jnp.float32),
            pltpu.SemaphoreType.DMA,
        ],
        compiler_params=pltpu.CompilerParams(use_tc_tiling_on_sc=False),
    )
    def gather_kernel(idx_hbm, table_hbm, out_hbm, idx_v, rows_v, gsem):
        wid = lax.axis_index("s") * info.num_cores + lax.axis_index("c")
        base = wid * chunks_per_w * CHUNK
        pltpu.sync_copy(idx_hbm.at[wid], idx_v)

        def body(j, _):
            pltpu.async_copy(table_hbm.at[idx_v.at[j]], rows_v.at[0], gsem).wait()
            pltpu.sync_copy(
                rows_v.at[0], out_hbm.at[pl.ds(base + j * CHUNK, CHUNK)]
            )
            return 0

        lax.fori_loop(0, chunks_per_w, body, 0)

    return gather_kernel, nw, chunks_per_w


def kernel(tok, table):
    b, l = tok.shape
    num_idx = b * l
    gather_kernel, nw, chunks_per_w = _make_gather(num_idx, DIM)
    idx = tok.reshape(nw, chunks_per_w, CHUNK)
    out = gather_kernel(idx, table)
    return out.reshape(b, l, DIM)


# speedup vs baseline: 1.1154x; 1.1154x over previous
import functools

import jax
import jax.numpy as jnp
from jax import lax
from jax.experimental import pallas as pl
from jax.experimental.pallas import tpu as pltpu
from jax.experimental.pallas import tpu_sc as plsc

DIM = 64
CHUNK = 128
S = 8
K = 4


@functools.cache
def _make_gather(num_idx: int, dim: int):
    info = plsc.get_sparse_core_info()
    nw = info.num_cores * info.num_subcores
    assert num_idx % (nw * CHUNK) == 0
    chunks_per_w = num_idx // (nw * CHUNK)
    mesh = plsc.VectorSubcoreMesh(core_axis_name="c", subcore_axis_name="s")

    @functools.partial(
        pl.kernel,
        mesh=mesh,
        out_type=jax.ShapeDtypeStruct((num_idx, dim), jnp.float32),
        scratch_types=[
            pltpu.VMEM((chunks_per_w, CHUNK), jnp.int32),
            pltpu.VMEM((S, CHUNK, dim), jnp.float32),
            pltpu.SemaphoreType.DMA,
            pltpu.SemaphoreType.DMA,
        ],
        compiler_params=pltpu.CompilerParams(use_tc_tiling_on_sc=False),
    )
    def gather_kernel(idx_hbm, table_hbm, out_hbm, idx_v, rows_v, gsem, osem):
        wid = lax.axis_index("s") * info.num_cores + lax.axis_index("c")
        base = wid * chunks_per_w * CHUNK
        pltpu.sync_copy(idx_hbm.at[wid], idx_v)

        for c in range(K):
            pltpu.async_copy(table_hbm.at[idx_v.at[c]], rows_v.at[c % S], gsem)

        @pl.loop(0, chunks_per_w, step=S)
        def outer(j0):
            for b in range(S):
                j = j0 + b
                jk = j + K
                bk = (b + K) % S

                @pl.when(jk < chunks_per_w)
                def _issue():
                    @pl.when(jk >= S)
                    def _drain():
                        pltpu.make_async_copy(
                            rows_v.at[bk], out_hbm.at[pl.ds(base, CHUNK)], osem
                        ).wait()

                    pltpu.async_copy(
                        table_hbm.at[idx_v.at[jk]], rows_v.at[bk], gsem
                    )

                pltpu.make_async_copy(
                    table_hbm.at[idx_v.at[j]], rows_v.at[b], gsem
                ).wait()
                pltpu.async_copy(
                    rows_v.at[b], out_hbm.at[pl.ds(base + j * CHUNK, CHUNK)], osem
                )

        for _ in range(S):
            pltpu.make_async_copy(
                rows_v.at[0], out_hbm.at[pl.ds(base, CHUNK)], osem
            ).wait()

    return gather_kernel, nw, chunks_per_w


def kernel(tok, table):
    b, l = tok.shape
    num_idx = b * l
    gather_kernel, nw, chunks_per_w = _make_gather(num_idx, DIM)
    idx = tok.reshape(nw, chunks_per_w, CHUNK)
    out = gather_kernel(idx, table)
    return out.reshape(b, l, DIM)
